# baseline (speedup 1.0000x reference)
import jax
import jax.numpy as jnp
from jax.experimental import pallas as pl
from jax.experimental.pallas import tpu as pltpu

KEEP_RATIO = 0.5


_NQ = 8


def _batch_copies(x_hbm, buf, sem, i):
    c = buf.shape[0]
    step = c // _NQ
    return [
        pltpu.make_async_copy(
            x_hbm.at[i, pl.ds(q * step, step)],
            buf.at[pl.ds(q * step, step)],
            sem.at[q],
        )
        for q in range(_NQ)
    ]


def _scores_body(x_hbm, o_ref, buf0, buf1, sem0, sem1):
    i = pl.program_id(0)
    n = pl.num_programs(0)

    @pl.when(i == 0)
    def _prime():
        for cp in _batch_copies(x_hbm, buf0, sem0, 0):
            cp.start()

    @pl.when(jnp.logical_and(i + 1 < n, (i + 1) % 2 == 0))
    def _pf_even():
        for cp in _batch_copies(x_hbm, buf0, sem0, i + 1):
            cp.start()

    @pl.when(jnp.logical_and(i + 1 < n, (i + 1) % 2 == 1))
    def _pf_odd():
        for cp in _batch_copies(x_hbm, buf1, sem1, i + 1):
            cp.start()

    @pl.when(i % 2 == 0)
    def _even():
        for cp in _batch_copies(x_hbm, buf0, sem0, i):
            cp.wait()
        o_ref[0, 0, :] = jnp.sum(jnp.abs(buf0[...]), axis=(1, 2))

    @pl.when(i % 2 == 1)
    def _odd():
        for cp in _batch_copies(x_hbm, buf1, sem1, i):
            cp.wait()
        o_ref[0, 0, :] = jnp.sum(jnp.abs(buf1[...]), axis=(1, 2))


def _mask_body(s_ref, o_ref):
    b, c = s_ref.shape
    k = max(1, min(c, int(round(c * KEEP_RATIO))))
    bits = jax.lax.bitcast_convert_type(s_ref[...], jnp.int32)
    col = jax.lax.broadcasted_iota(jnp.int32, (b, c), 1)

    def vsearch(_, carry):
        lo, hi = carry
        mid = lo + ((hi - lo + 1) >> 1)
        cnt = jnp.sum((bits >= mid).astype(jnp.int32), axis=1, keepdims=True)
        p = cnt >= k
        return jnp.where(p, mid, lo), jnp.where(p, hi, mid - 1)

    lo = jnp.zeros((b, 1), jnp.int32)
    hi = jnp.full((b, 1), 0x7F800000, jnp.int32)
    t, _ = jax.lax.fori_loop(0, 31, vsearch, (lo, hi))

    gt = bits > t
    eq = bits == t
    need_eq = k - jnp.sum(gt.astype(jnp.int32), axis=1, keepdims=True)

    def isearch(_, carry):
        lo2, hi2 = carry
        mid = (lo2 + hi2) >> 1
        cnt = jnp.sum((eq & (col <= mid)).astype(jnp.int32), axis=1,
                      keepdims=True)
        p = cnt >= need_eq
        return jnp.where(p, lo2, mid + 1), jnp.where(p, mid, hi2)

    lo2 = jnp.zeros((b, 1), jnp.int32)
    hi2 = jnp.full((b, 1), c - 1, jnp.int32)
    m, _ = jax.lax.fori_loop(0, 10, isearch, (lo2, hi2))

    o_ref[...] = (gt | (eq & (col <= m))).astype(jnp.float32)


def kernel(features, enabled):
    b, c, h, w = features.shape

    scores3 = pl.pallas_call(
        _scores_body,
        grid=(b,),
        in_specs=[pl.BlockSpec(memory_space=pltpu.HBM)],
        out_specs=pl.BlockSpec((1, 1, c), lambda i: (i, 0, 0)),
        out_shape=jax.ShapeDtypeStruct((b, 1, c), jnp.float32),
        scratch_shapes=[
            pltpu.VMEM((c, h, w), jnp.float32),
            pltpu.VMEM((c, h, w), jnp.float32),
            pltpu.SemaphoreType.DMA((_NQ,)),
            pltpu.SemaphoreType.DMA((_NQ,)),
        ],
        compiler_params=pltpu.CompilerParams(
            dimension_semantics=("arbitrary",)),
    )(features)
    scores = scores3.reshape(b, c)

    mask = pl.pallas_call(
        _mask_body,
        out_shape=jax.ShapeDtypeStruct((b, c), jnp.float32),
    )(scores)

    gated = mask.reshape(b, c, 1, 1)
    return jnp.where(jnp.asarray(enabled) != 0, gated,
                     jnp.ones_like(gated))
